# Initial kernel scaffold; baseline (speedup 1.0000x reference)
import jax
import jax.numpy as jnp
from jax.experimental import pallas as pl


def kernel(x, mem_k, mem_v, Wq, Wkv, Wo, scale_param):
    raise NotImplementedError("write your pallas kernel here")


# speedup vs baseline: 15.4608x; 15.4608x over previous
import functools

import jax
import jax.numpy as jnp
from jax.experimental import pallas as pl
from jax.experimental.pallas import tpu as pltpu

B, N, D, H, DH, M, K = 1, 2048, 1024, 16, 64, 4096, 32

_PREC = jax.lax.Precision.DEFAULT
_PREC_EXACT = jax.lax.Precision.HIGHEST
_BISECT_ITERS = 26
_NEG = -1e30


def _qproj_kernel(x_ref, wq_ref, q_ref):
    q = jax.lax.dot_general(x_ref[...], wq_ref[0], (((1,), (0,)), ((), ())),
                            preferred_element_type=jnp.float32, precision=_PREC)
    norm = jnp.sqrt(jnp.sum(q * q, axis=-1, keepdims=True))
    q_ref[0] = q / jnp.maximum(norm, 1e-12)


def _kv_kernel(x_ref, wkv_ref, memk_ref, k_ref, v_ref, memkn_ref):
    kv = jax.lax.dot_general(x_ref[...], wkv_ref[...], (((1,), (0,)), ((), ())),
                             preferred_element_type=jnp.float32, precision=_PREC)
    k = kv[:, :DH]
    v = kv[:, DH:]
    knorm = jnp.sqrt(jnp.sum(k * k, axis=-1, keepdims=True))
    k_ref[...] = k / jnp.maximum(knorm, 1e-12)
    v_ref[...] = v
    mk = memk_ref[...]
    mnorm = jnp.sqrt(jnp.sum(mk * mk, axis=-1, keepdims=True))
    memkn_ref[...] = mk / jnp.maximum(mnorm, 1e-12)


def _attn_kernel(scale_ref, q_ref, kn_ref, v_ref, mk_ref, mv_ref, o_ref, *, bq):
    h = pl.program_id(0)
    i = pl.program_id(1)
    q = q_ref[0]
    sc = scale_ref[h]

    s_sel = jax.lax.dot_general(q, mk_ref[...], (((1,), (1,)), ((), ())),
                                preferred_element_type=jnp.float32, precision=_PREC)

    lo0 = jnp.min(s_sel, axis=-1, keepdims=True) - 1.0
    hi0 = jnp.max(s_sel, axis=-1, keepdims=True) + 1.0

    def body(_, carry):
        lo, hi = carry
        mid = 0.5 * (lo + hi)
        cnt = jnp.sum((s_sel >= mid).astype(jnp.float32), axis=-1, keepdims=True)
        ge = cnt >= K
        return jnp.where(ge, mid, lo), jnp.where(ge, hi, mid)

    lo, _ = jax.lax.fori_loop(0, _BISECT_ITERS, body, (lo0, hi0))
    sel = s_sel >= lo

    s_mem = jax.lax.dot_general(q, mk_ref[...], (((1,), (1,)), ((), ())),
                                preferred_element_type=jnp.float32,
                                precision=_PREC_EXACT)

    s_loc = jax.lax.dot_general(q, kn_ref[...], (((1,), (1,)), ((), ())),
                                preferred_element_type=jnp.float32, precision=_PREC)
    row = i * bq + jax.lax.broadcasted_iota(jnp.int32, (bq, N), 0)
    col = jax.lax.broadcasted_iota(jnp.int32, (bq, N), 1)
    causal = col <= row

    l_mem = jnp.where(sel, sc * s_mem, _NEG)
    l_loc = jnp.where(causal, sc * s_loc, _NEG)
    m = jnp.maximum(jnp.max(l_mem, axis=-1, keepdims=True),
                    jnp.max(l_loc, axis=-1, keepdims=True))
    p_mem = jnp.where(sel, jnp.exp(l_mem - m), 0.0)
    p_loc = jnp.where(causal, jnp.exp(l_loc - m), 0.0)
    denom = (jnp.sum(p_mem, axis=-1, keepdims=True)
             + jnp.sum(p_loc, axis=-1, keepdims=True))
    o = (jax.lax.dot_general(p_mem, mv_ref[...], (((1,), (0,)), ((), ())),
                             preferred_element_type=jnp.float32,
                             precision=_PREC_EXACT)
         + jax.lax.dot_general(p_loc, v_ref[...], (((1,), (0,)), ((), ())),
                               preferred_element_type=jnp.float32, precision=_PREC))
    o_ref[0] = o / denom


def _oproj_kernel(y_ref, wo_ref, out_ref):
    out_ref[...] = jax.lax.dot_general(
        y_ref[...], wo_ref[...], (((1,), (0,)), ((), ())),
        preferred_element_type=jnp.float32, precision=_PREC)


def kernel(x, mem_k, mem_v, Wq, Wkv, Wo, scale_param):
    x2 = x[0]
    mk = mem_k[0]
    mv = mem_v[0]
    scale = jnp.exp(scale_param).reshape(H)
    wq3 = Wq.reshape(D, H, DH).transpose(1, 0, 2)

    q_n = pl.pallas_call(
        _qproj_kernel,
        grid=(H,),
        in_specs=[
            pl.BlockSpec((N, D), lambda h: (0, 0)),
            pl.BlockSpec((1, D, DH), lambda h: (h, 0, 0)),
        ],
        out_specs=pl.BlockSpec((1, N, DH), lambda h: (h, 0, 0)),
        out_shape=jax.ShapeDtypeStruct((H, N, DH), jnp.float32),
    )(x2, wq3)

    k_n, v, mk_n = pl.pallas_call(
        _kv_kernel,
        out_shape=(
            jax.ShapeDtypeStruct((N, DH), jnp.float32),
            jax.ShapeDtypeStruct((N, DH), jnp.float32),
            jax.ShapeDtypeStruct((M, DH), jnp.float32),
        ),
    )(x2, Wkv, mk)

    bq = 256
    attn = pl.pallas_call(
        functools.partial(_attn_kernel, bq=bq),
        grid=(H, N // bq),
        in_specs=[
            pl.BlockSpec(memory_space=pltpu.SMEM),
            pl.BlockSpec((1, bq, DH), lambda h, i: (h, i, 0)),
            pl.BlockSpec((N, DH), lambda h, i: (0, 0)),
            pl.BlockSpec((N, DH), lambda h, i: (0, 0)),
            pl.BlockSpec((M, DH), lambda h, i: (0, 0)),
            pl.BlockSpec((M, DH), lambda h, i: (0, 0)),
        ],
        out_specs=pl.BlockSpec((1, bq, DH), lambda h, i: (h, i, 0)),
        out_shape=jax.ShapeDtypeStruct((H, N, DH), jnp.float32),
    )(scale, q_n, k_n, v, mk_n, mv)

    y = attn.transpose(1, 0, 2).reshape(N, H * DH)

    bo = 256
    out = pl.pallas_call(
        _oproj_kernel,
        grid=(N // bo,),
        in_specs=[
            pl.BlockSpec((bo, H * DH), lambda i: (i, 0)),
            pl.BlockSpec((H * DH, D), lambda i: (0, 0)),
        ],
        out_specs=pl.BlockSpec((bo, D), lambda i: (i, 0)),
        out_shape=jax.ShapeDtypeStruct((N, D), jnp.float32),
    )(y, Wo)

    return out.reshape(B, N, D)
